# baseline (speedup 1.0000x reference)
import functools

import jax
import jax.numpy as jnp
from jax import lax
from jax.experimental import pallas as pl
from jax.experimental.pallas import tpu as pltpu
from jax.experimental.pallas import tpu_sc as plsc

N = 10000
E = 320000
D = 128

NC = 2
NS = 16
NW = NC * NS
L = 16

NP = 10240
TRASH = NP - 1
E_W = E // NW
EP_W = 10368
B = 64
NCHUNK = EP_W // B
ROWS_S = NP // NS
NROW = NP // D

R = 3
NGROUP = NCHUNK // R


def _wid(c, s):
    return s * NC + c


def _split_idx(v):
    return [lax.shift_right_logical(v, 7), jnp.bitwise_and(v, 127)]


def _zero2d(ref):
    zero = jnp.zeros((L,), jnp.float32)

    def zbody(i, _):
        for cc in range(D // L):
            ref[i, pl.ds(cc * L, L)] = zero
        return ()

    lax.fori_loop(0, NROW, zbody, ())


def _sc_deg_body(dst_hbm, out_hbm, hist_v, idx_v):
    c = lax.axis_index("c")
    s = lax.axis_index("s")
    w = _wid(c, s)

    _zero2d(hist_v)
    pltpu.sync_copy(dst_hbm.at[pl.ds(w * EP_W, EP_W)], idx_v)

    ones = jnp.ones((L,), jnp.float32)

    def body(i, _):
        didx = idx_v[pl.ds(i * L, L)]
        plsc.addupdate_scatter(hist_v, _split_idx(didx), ones)
        return ()

    lax.fori_loop(0, EP_W // L, body, ())
    pltpu.sync_copy(hist_v, out_hbm.at[w])


def _ring(table_hbm, dst_hbm, ebase, sidx_all, dss, rows, gsem, ssem, dsem,
          acc_sh, stage_fn=None, scale_fn=None):
    def g_desc(j, r):
        return pltpu.make_async_copy(
            table_hbm.at[sidx_all.at[pl.ds(j * B, B)]], rows[r], gsem[r])

    def d_desc(j, r):
        return pltpu.make_async_copy(dst_hbm.at[pl.ds(ebase + j * B, B)],
                                     dss[r], dsem[r])

    def fire(j, r):
        d_desc(j, r).start()
        if stage_fn is not None:
            stage_fn(j, r)
        g_desc(j, r).start()

    def s_desc(r):
        return pltpu.make_async_copy(rows[r], acc_sh.at[dss[r]], ssem[r])

    for r in range(R - 1):
        fire(r, r)
    plsc.subcore_barrier()

    def group(jg, _):
        for r in range(R):
            j = jg * R + r
            g_desc(j, r).wait()
            if scale_fn is not None:
                scale_fn(j, r)
            d_desc(j, r).wait()
            pltpu.async_copy(rows[r], acc_sh.at[dss[r]], ssem[r], add=True)
            rn = (r + R - 1) % R
            jn = j + R - 1

            @pl.when(jn < NCHUNK)
            def _():
                @pl.when(j >= 1)
                def _():
                    s_desc(rn).wait()

                fire(jn, rn)
        return ()

    lax.fori_loop(0, NGROUP, group, ())
    for r in range(R):
        s_desc(r).wait()
    plsc.subcore_barrier()


def _sc_pass1_body(xs_hbm, src_hbm, dst_hbm, zeros_hbm, out_hbm,
                   acc_sh, sidx_all, ds0, ds1, ds2,
                   rows0, rows1, rows2, gs0, gs1, gs2, ss0, ss1, ss2,
                   dm0, dm1, dm2):
    c = lax.axis_index("c")
    s = lax.axis_index("s")
    w = _wid(c, s)
    ebase = w * EP_W

    pltpu.sync_copy(src_hbm.at[pl.ds(ebase, EP_W)], sidx_all)
    pltpu.sync_copy(zeros_hbm.at[pl.ds(s * ROWS_S, ROWS_S)],
                    acc_sh.at[pl.ds(s * ROWS_S, ROWS_S)])

    _ring(xs_hbm, dst_hbm, ebase, sidx_all, [ds0, ds1, ds2],
          [rows0, rows1, rows2], [gs0, gs1, gs2], [ss0, ss1, ss2],
          [dm0, dm1, dm2], acc_sh)

    pltpu.sync_copy(acc_sh.at[pl.ds(s * ROWS_S, ROWS_S)],
                    out_hbm.at[pl.ds(c * NP + s * ROWS_S, ROWS_S)])


def _sc_pass2a_body(src_hbm, dst_hbm, asrc_hbm, adst_hbm, shift_hbm,
                    ex_hbm, den_hbm,
                    asrc_v, adst_v, den_v, sidx_v, didx_v, ex_v, shift_v):
    c = lax.axis_index("c")
    s = lax.axis_index("s")
    w = _wid(c, s)
    ebase = w * EP_W

    pltpu.sync_copy(src_hbm.at[pl.ds(ebase, EP_W)], sidx_v)
    pltpu.sync_copy(dst_hbm.at[pl.ds(ebase, EP_W)], didx_v)
    pltpu.sync_copy(asrc_hbm, asrc_v)
    pltpu.sync_copy(adst_hbm, adst_v)
    pltpu.sync_copy(shift_hbm, shift_v)
    _zero2d(den_v)

    shvec = shift_v[...]

    def body(i, _):
        sv = sidx_v[pl.ds(i * L, L)]
        dv = didx_v[pl.ds(i * L, L)]
        av = plsc.load_gather(asrc_v, [sv])
        bv = plsc.load_gather(adst_v, [dv])
        v = av + bv
        al = jnp.where(v > 0, v, 0.2 * v)
        ex = jnp.exp(al - shvec)
        plsc.addupdate_scatter(den_v, _split_idx(dv), ex)
        ex_v[pl.ds(i * L, L)] = ex
        return ()

    lax.fori_loop(0, EP_W // L, body, ())
    pltpu.sync_copy(ex_v, ex_hbm.at[pl.ds(ebase, EP_W)])
    pltpu.sync_copy(den_v, den_hbm.at[w])


def _sc_pass2b_body(xg_hbm, src_hbm, dst_hbm, ex_hbm, zeros_hbm, num_hbm,
                    acc_sh, sidx_all, ds0, ds1, ds2,
                    eb0, eb1, eb2, rows0, rows1, rows2,
                    gs0, gs1, gs2, ss0, ss1, ss2, dm0, dm1, dm2,
                    es0, es1, es2):
    c = lax.axis_index("c")
    s = lax.axis_index("s")
    w = _wid(c, s)
    ebase = w * EP_W

    pltpu.sync_copy(src_hbm.at[pl.ds(ebase, EP_W)], sidx_all)
    pltpu.sync_copy(zeros_hbm.at[pl.ds(s * ROWS_S, ROWS_S)],
                    acc_sh.at[pl.ds(s * ROWS_S, ROWS_S)])

    rows = [rows0, rows1, rows2]
    exs = [eb0, eb1, eb2]
    esem = [es0, es1, es2]

    def e_desc(j, r):
        return pltpu.make_async_copy(ex_hbm.at[pl.ds(ebase + j * B, B)],
                                     exs[r], esem[r])

    def stage(j, r):
        e_desc(j, r).start()

    def scale(j, r):
        e_desc(j, r).wait()

        def kbody(k, _):
            for e in range(L):
                row = k * L + e
                exb = plsc.load_gather(exs[r], [jnp.full((L,), row,
                                                         jnp.int32)])
                for cc in range(D // L):
                    sl = pl.ds(cc * L, L)
                    rows[r][row, sl] = rows[r][row, sl] * exb
            return ()

        lax.fori_loop(0, B // L, kbody, ())

    _ring(xg_hbm, dst_hbm, ebase, sidx_all, [ds0, ds1, ds2], rows,
          [gs0, gs1, gs2], [ss0, ss1, ss2], [dm0, dm1, dm2], acc_sh,
          stage_fn=stage, scale_fn=scale)

    pltpu.sync_copy(acc_sh.at[pl.ds(s * ROWS_S, ROWS_S)],
                    num_hbm.at[pl.ds(c * NP + s * ROWS_S, ROWS_S)])


@functools.cache
def _sc_kernels():
    mesh = plsc.VectorSubcoreMesh(core_axis_name="c", subcore_axis_name="s",
                                  num_cores=NC, num_subcores=NS)
    cp = pltpu.CompilerParams(needs_layout_passes=False)
    f32 = jnp.float32
    i32 = jnp.int32
    sc_deg = pl.kernel(
        _sc_deg_body,
        out_type=jax.ShapeDtypeStruct((NW, NROW, D), f32),
        mesh=mesh,
        scratch_types=[
            pltpu.VMEM((NROW, D), f32),
            pltpu.VMEM((EP_W,), i32),
        ],
        compiler_params=cp,
    )
    idx_all = [pltpu.VMEM((EP_W,), i32)]
    dss_ring = [pltpu.VMEM((B,), i32)] * R
    rows_ring = [pltpu.VMEM((B, D), f32)] * R
    sems = [pltpu.SemaphoreType.DMA] * (3 * R)
    sc_pass1 = pl.kernel(
        _sc_pass1_body,
        out_type=jax.ShapeDtypeStruct((NC * NP, D), f32),
        mesh=mesh,
        scratch_types=[pltpu.VMEM_SHARED((NP, D), f32)]
        + idx_all + dss_ring + rows_ring + sems,
        compiler_params=cp,
    )
    sc_pass2a = pl.kernel(
        _sc_pass2a_body,
        out_type=[
            jax.ShapeDtypeStruct((NW * EP_W,), f32),
            jax.ShapeDtypeStruct((NW, NROW, D), f32),
        ],
        mesh=mesh,
        scratch_types=[
            pltpu.VMEM((NP,), f32),
            pltpu.VMEM((NP,), f32),
            pltpu.VMEM((NROW, D), f32),
            pltpu.VMEM((EP_W,), i32),
            pltpu.VMEM((EP_W,), i32),
            pltpu.VMEM((EP_W,), f32),
            pltpu.VMEM((L,), f32),
        ],
        compiler_params=cp,
    )
    sc_pass2b = pl.kernel(
        _sc_pass2b_body,
        out_type=jax.ShapeDtypeStruct((NC * NP, D), f32),
        mesh=mesh,
        scratch_types=[pltpu.VMEM_SHARED((NP, D), f32)]
        + idx_all + dss_ring + [pltpu.VMEM((B,), f32)] * R + rows_ring
        + sems + [pltpu.SemaphoreType.DMA] * R,
        compiler_params=cp,
    )
    return sc_deg, sc_pass1, sc_pass2a, sc_pass2b


def _tc1_body(z_ref, w1_ref, degp_ref, xs_ref, dis_ref):
    deg = jnp.sum(degp_ref[...], axis=1, keepdims=True) + 1.0
    dis = lax.rsqrt(deg)
    x = jnp.dot(z_ref[...], w1_ref[...], preferred_element_type=jnp.float32)
    xs_ref[...] = x * dis
    dis_ref[...] = dis


def _tc2_body(acc_ref, xs_ref, dis_ref, b1_ref, pa_ref, w2_ref,
              asv_ref, adv_ref, xg_ref, asrc_ref, adst_ref, shift_ref,
              exself_ref):
    ssum = acc_ref[0:N] + acc_ref[NP:NP + N] + xs_ref[...]
    h1 = dis_ref[...] * ssum + b1_ref[...]
    h1 = jnp.where(h1 >= 0, h1, pa_ref[...] * h1)
    xg = jnp.dot(h1, w2_ref[...], preferred_element_type=jnp.float32)
    xg_ref[...] = xg
    a_src = jnp.sum(xg * asv_ref[...], axis=1, keepdims=True)
    a_dst = jnp.sum(xg * adv_ref[...], axis=1, keepdims=True)
    asrc_ref[...] = a_src
    adst_ref[...] = a_dst
    vmax = jnp.max(a_src) + jnp.max(a_dst)
    shift = jnp.where(vmax > 0, vmax, 0.2 * vmax)
    shift_ref[...] = jnp.full((1, L), shift, jnp.float32)
    v_self = a_src + a_dst
    al_self = jnp.where(v_self > 0, v_self, 0.2 * v_self)
    exself_ref[...] = jnp.exp(al_self - shift)


def _tc3_body(num_ref, xg_ref, denp_ref, exself_ref, b2_ref, out_ref):
    exself = exself_ref[...]
    numt = num_ref[0:N] + num_ref[NP:NP + N] + exself * xg_ref[...]
    dent = jnp.sum(denp_ref[...], axis=1, keepdims=True) + exself + 1e-16
    out_ref[...] = numt / dent + b2_ref[...]


_tc1 = pl.pallas_call(
    _tc1_body,
    out_shape=[jax.ShapeDtypeStruct((N, D), jnp.float32),
               jax.ShapeDtypeStruct((N, 1), jnp.float32)],
)

_tc2 = pl.pallas_call(
    _tc2_body,
    out_shape=[jax.ShapeDtypeStruct((N, D), jnp.float32),
               jax.ShapeDtypeStruct((N, 1), jnp.float32),
               jax.ShapeDtypeStruct((N, 1), jnp.float32),
               jax.ShapeDtypeStruct((1, L), jnp.float32),
               jax.ShapeDtypeStruct((N, 1), jnp.float32)],
)

_tc3 = pl.pallas_call(
    _tc3_body,
    out_shape=jax.ShapeDtypeStruct((N, D), jnp.float32),
)


def _pad_edges(v, fill):
    v2 = v.reshape(NW, E_W)
    return jnp.pad(v2, ((0, 0), (0, EP_W - E_W)),
                   constant_values=fill).reshape(NW * EP_W)


def kernel(z, edge_index, W1, b1, prelu_a, W2, att_src, att_dst, b2):
    src = _pad_edges(edge_index[0].astype(jnp.int32), 0)
    dst = _pad_edges(edge_index[1].astype(jnp.int32), TRASH)
    zeros = jnp.zeros((NP, D), jnp.float32)

    sc_deg, sc_pass1, sc_pass2a, sc_pass2b = _sc_kernels()

    degp = sc_deg(dst)
    degp_t = degp.reshape(NW, NP).T[:N]
    xs, dis = _tc1(z, W1, degp_t)
    acc = sc_pass1(xs, src, dst, zeros)
    xg, a_src, a_dst, shift, exself = _tc2(
        acc, xs, dis, b1.reshape(1, D), prelu_a.reshape(1, D), W2,
        att_src.reshape(1, D), att_dst.reshape(1, D))
    asrc_p = jnp.pad(a_src.reshape(N), (0, NP - N))
    adst_p = jnp.pad(a_dst.reshape(N), (0, NP - N))
    ex, denp = sc_pass2a(src, dst, asrc_p, adst_p, shift.reshape(L))
    num = sc_pass2b(xg, src, dst, ex, zeros)
    denp_t = denp.reshape(NW, NP).T[:N]
    out = _tc3(num, xg, denp_t, exself, b2.reshape(1, D))
    return out


# speedup vs baseline: 1.0136x; 1.0136x over previous
import functools

import jax
import jax.numpy as jnp
from jax import lax
from jax.experimental import pallas as pl
from jax.experimental.pallas import tpu as pltpu
from jax.experimental.pallas import tpu_sc as plsc

N = 10000
E = 320000
D = 128

NC = 2
NS = 16
NW = NC * NS
L = 16

NP = 10240
TRASH = NP - 1
E_W = E // NW
EP_W = 10368
B = 96
NCHUNK = EP_W // B
ROWS_S = NP // NS
NROW = NP // D

R = 3
NGROUP = NCHUNK // R


def _wid(c, s):
    return s * NC + c


def _split_idx(v):
    return [lax.shift_right_logical(v, 7), jnp.bitwise_and(v, 127)]


def _zero2d(ref):
    zero = jnp.zeros((L,), jnp.float32)

    def zbody(i, _):
        for cc in range(D // L):
            ref[i, pl.ds(cc * L, L)] = zero
        return ()

    lax.fori_loop(0, NROW, zbody, ())


def _sc_deg_body(dst_hbm, out_hbm, hist_v, idx_v):
    c = lax.axis_index("c")
    s = lax.axis_index("s")
    w = _wid(c, s)

    _zero2d(hist_v)
    pltpu.sync_copy(dst_hbm.at[pl.ds(w * EP_W, EP_W)], idx_v)

    ones = jnp.ones((L,), jnp.float32)

    def body(i, _):
        didx = idx_v[pl.ds(i * L, L)]
        plsc.addupdate_scatter(hist_v, _split_idx(didx), ones)
        return ()

    lax.fori_loop(0, EP_W // L, body, ())
    pltpu.sync_copy(hist_v, out_hbm.at[w])


def _ring(table_hbm, dst_hbm, ebase, sidx_all, dss, rows, gsem, ssem, dsem,
          acc_sh, stage_fn=None, scale_fn=None):
    def g_desc(j, r):
        return pltpu.make_async_copy(
            table_hbm.at[sidx_all.at[pl.ds(j * B, B)]], rows[r], gsem[r])

    def d_desc(j, r):
        return pltpu.make_async_copy(dst_hbm.at[pl.ds(ebase + j * B, B)],
                                     dss[r], dsem[r])

    def fire(j, r):
        d_desc(j, r).start()
        if stage_fn is not None:
            stage_fn(j, r)
        g_desc(j, r).start()

    def s_desc(r):
        return pltpu.make_async_copy(rows[r], acc_sh.at[dss[r]], ssem[r])

    for r in range(R - 1):
        fire(r, r)
    plsc.subcore_barrier()

    def group(jg, _):
        for r in range(R):
            j = jg * R + r
            g_desc(j, r).wait()
            if scale_fn is not None:
                scale_fn(j, r)
            d_desc(j, r).wait()
            pltpu.async_copy(rows[r], acc_sh.at[dss[r]], ssem[r], add=True)
            rn = (r + R - 1) % R
            jn = j + R - 1

            @pl.when(jn < NCHUNK)
            def _():
                @pl.when(j >= 1)
                def _():
                    s_desc(rn).wait()

                fire(jn, rn)
        return ()

    lax.fori_loop(0, NGROUP, group, ())
    for r in range(R):
        s_desc(r).wait()
    plsc.subcore_barrier()


def _sc_pass1_body(xs_hbm, src_hbm, dst_hbm, zeros_hbm, out_hbm,
                   acc_sh, sidx_all, ds0, ds1, ds2,
                   rows0, rows1, rows2, gs0, gs1, gs2, ss0, ss1, ss2,
                   dm0, dm1, dm2):
    c = lax.axis_index("c")
    s = lax.axis_index("s")
    w = _wid(c, s)
    ebase = w * EP_W

    pltpu.sync_copy(src_hbm.at[pl.ds(ebase, EP_W)], sidx_all)
    pltpu.sync_copy(zeros_hbm.at[pl.ds(s * ROWS_S, ROWS_S)],
                    acc_sh.at[pl.ds(s * ROWS_S, ROWS_S)])

    _ring(xs_hbm, dst_hbm, ebase, sidx_all, [ds0, ds1, ds2],
          [rows0, rows1, rows2], [gs0, gs1, gs2], [ss0, ss1, ss2],
          [dm0, dm1, dm2], acc_sh)

    pltpu.sync_copy(acc_sh.at[pl.ds(s * ROWS_S, ROWS_S)],
                    out_hbm.at[pl.ds(c * NP + s * ROWS_S, ROWS_S)])


def _sc_pass2a_body(src_hbm, dst_hbm, asrc_hbm, adst_hbm, shift_hbm,
                    ex_hbm, den_hbm,
                    asrc_v, adst_v, den_v, sidx_v, didx_v, ex_v, shift_v):
    c = lax.axis_index("c")
    s = lax.axis_index("s")
    w = _wid(c, s)
    ebase = w * EP_W

    pltpu.sync_copy(src_hbm.at[pl.ds(ebase, EP_W)], sidx_v)
    pltpu.sync_copy(dst_hbm.at[pl.ds(ebase, EP_W)], didx_v)
    pltpu.sync_copy(asrc_hbm, asrc_v)
    pltpu.sync_copy(adst_hbm, adst_v)
    pltpu.sync_copy(shift_hbm, shift_v)
    _zero2d(den_v)

    shvec = shift_v[...]

    def body(i, _):
        sv = sidx_v[pl.ds(i * L, L)]
        dv = didx_v[pl.ds(i * L, L)]
        av = plsc.load_gather(asrc_v, [sv])
        bv = plsc.load_gather(adst_v, [dv])
        v = av + bv
        al = jnp.where(v > 0, v, 0.2 * v)
        ex = jnp.exp(al - shvec)
        plsc.addupdate_scatter(den_v, _split_idx(dv), ex)
        ex_v[pl.ds(i * L, L)] = ex
        return ()

    lax.fori_loop(0, EP_W // L, body, ())
    pltpu.sync_copy(ex_v, ex_hbm.at[pl.ds(ebase, EP_W)])
    pltpu.sync_copy(den_v, den_hbm.at[w])


def _sc_pass2b_body(xg_hbm, src_hbm, dst_hbm, ex_hbm, zeros_hbm, num_hbm,
                    acc_sh, sidx_all, ds0, ds1, ds2,
                    eb0, eb1, eb2, rows0, rows1, rows2,
                    gs0, gs1, gs2, ss0, ss1, ss2, dm0, dm1, dm2,
                    es0, es1, es2):
    c = lax.axis_index("c")
    s = lax.axis_index("s")
    w = _wid(c, s)
    ebase = w * EP_W

    pltpu.sync_copy(src_hbm.at[pl.ds(ebase, EP_W)], sidx_all)
    pltpu.sync_copy(zeros_hbm.at[pl.ds(s * ROWS_S, ROWS_S)],
                    acc_sh.at[pl.ds(s * ROWS_S, ROWS_S)])

    rows = [rows0, rows1, rows2]
    exs = [eb0, eb1, eb2]
    esem = [es0, es1, es2]

    def e_desc(j, r):
        return pltpu.make_async_copy(ex_hbm.at[pl.ds(ebase + j * B, B)],
                                     exs[r], esem[r])

    def stage(j, r):
        e_desc(j, r).start()

    def scale(j, r):
        e_desc(j, r).wait()

        def kbody(k, _):
            for e in range(L):
                row = k * L + e
                exb = plsc.load_gather(exs[r], [jnp.full((L,), row,
                                                         jnp.int32)])
                for cc in range(D // L):
                    sl = pl.ds(cc * L, L)
                    rows[r][row, sl] = rows[r][row, sl] * exb
            return ()

        lax.fori_loop(0, B // L, kbody, ())

    _ring(xg_hbm, dst_hbm, ebase, sidx_all, [ds0, ds1, ds2], rows,
          [gs0, gs1, gs2], [ss0, ss1, ss2], [dm0, dm1, dm2], acc_sh,
          stage_fn=stage, scale_fn=scale)

    pltpu.sync_copy(acc_sh.at[pl.ds(s * ROWS_S, ROWS_S)],
                    num_hbm.at[pl.ds(c * NP + s * ROWS_S, ROWS_S)])


@functools.cache
def _sc_kernels():
    mesh = plsc.VectorSubcoreMesh(core_axis_name="c", subcore_axis_name="s",
                                  num_cores=NC, num_subcores=NS)
    cp = pltpu.CompilerParams(needs_layout_passes=False)
    f32 = jnp.float32
    i32 = jnp.int32
    sc_deg = pl.kernel(
        _sc_deg_body,
        out_type=jax.ShapeDtypeStruct((NW, NROW, D), f32),
        mesh=mesh,
        scratch_types=[
            pltpu.VMEM((NROW, D), f32),
            pltpu.VMEM((EP_W,), i32),
        ],
        compiler_params=cp,
    )
    idx_all = [pltpu.VMEM((EP_W,), i32)]
    dss_ring = [pltpu.VMEM((B,), i32)] * R
    rows_ring = [pltpu.VMEM((B, D), f32)] * R
    sems = [pltpu.SemaphoreType.DMA] * (3 * R)
    sc_pass1 = pl.kernel(
        _sc_pass1_body,
        out_type=jax.ShapeDtypeStruct((NC * NP, D), f32),
        mesh=mesh,
        scratch_types=[pltpu.VMEM_SHARED((NP, D), f32)]
        + idx_all + dss_ring + rows_ring + sems,
        compiler_params=cp,
    )
    sc_pass2a = pl.kernel(
        _sc_pass2a_body,
        out_type=[
            jax.ShapeDtypeStruct((NW * EP_W,), f32),
            jax.ShapeDtypeStruct((NW, NROW, D), f32),
        ],
        mesh=mesh,
        scratch_types=[
            pltpu.VMEM((NP,), f32),
            pltpu.VMEM((NP,), f32),
            pltpu.VMEM((NROW, D), f32),
            pltpu.VMEM((EP_W,), i32),
            pltpu.VMEM((EP_W,), i32),
            pltpu.VMEM((EP_W,), f32),
            pltpu.VMEM((L,), f32),
        ],
        compiler_params=cp,
    )
    sc_pass2b = pl.kernel(
        _sc_pass2b_body,
        out_type=jax.ShapeDtypeStruct((NC * NP, D), f32),
        mesh=mesh,
        scratch_types=[pltpu.VMEM_SHARED((NP, D), f32)]
        + idx_all + dss_ring + [pltpu.VMEM((B,), f32)] * R + rows_ring
        + sems + [pltpu.SemaphoreType.DMA] * R,
        compiler_params=cp,
    )
    return sc_deg, sc_pass1, sc_pass2a, sc_pass2b


def _tc1_body(z_ref, w1_ref, degp_ref, xs_ref, dis_ref):
    deg = jnp.sum(degp_ref[...], axis=1, keepdims=True) + 1.0
    dis = lax.rsqrt(deg)
    x = jnp.dot(z_ref[...], w1_ref[...], preferred_element_type=jnp.float32)
    xs_ref[...] = x * dis
    dis_ref[...] = dis


def _tc2_body(acc_ref, xs_ref, dis_ref, b1_ref, pa_ref, w2_ref,
              asv_ref, adv_ref, xg_ref, asrc_ref, adst_ref, shift_ref,
              exself_ref):
    ssum = acc_ref[0:N] + acc_ref[NP:NP + N] + xs_ref[...]
    h1 = dis_ref[...] * ssum + b1_ref[...]
    h1 = jnp.where(h1 >= 0, h1, pa_ref[...] * h1)
    xg = jnp.dot(h1, w2_ref[...], preferred_element_type=jnp.float32)
    xg_ref[...] = xg
    a_src = jnp.sum(xg * asv_ref[...], axis=1, keepdims=True)
    a_dst = jnp.sum(xg * adv_ref[...], axis=1, keepdims=True)
    asrc_ref[...] = a_src
    adst_ref[...] = a_dst
    vmax = jnp.max(a_src) + jnp.max(a_dst)
    shift = jnp.where(vmax > 0, vmax, 0.2 * vmax)
    shift_ref[...] = jnp.full((1, L), shift, jnp.float32)
    v_self = a_src + a_dst
    al_self = jnp.where(v_self > 0, v_self, 0.2 * v_self)
    exself_ref[...] = jnp.exp(al_self - shift)


def _tc3_body(num_ref, xg_ref, denp_ref, exself_ref, b2_ref, out_ref):
    exself = exself_ref[...]
    numt = num_ref[0:N] + num_ref[NP:NP + N] + exself * xg_ref[...]
    dent = jnp.sum(denp_ref[...], axis=1, keepdims=True) + exself + 1e-16
    out_ref[...] = numt / dent + b2_ref[...]


_tc1 = pl.pallas_call(
    _tc1_body,
    out_shape=[jax.ShapeDtypeStruct((N, D), jnp.float32),
               jax.ShapeDtypeStruct((N, 1), jnp.float32)],
)

_tc2 = pl.pallas_call(
    _tc2_body,
    out_shape=[jax.ShapeDtypeStruct((N, D), jnp.float32),
               jax.ShapeDtypeStruct((N, 1), jnp.float32),
               jax.ShapeDtypeStruct((N, 1), jnp.float32),
               jax.ShapeDtypeStruct((1, L), jnp.float32),
               jax.ShapeDtypeStruct((N, 1), jnp.float32)],
)

_tc3 = pl.pallas_call(
    _tc3_body,
    out_shape=jax.ShapeDtypeStruct((N, D), jnp.float32),
)


def _pad_edges(v, fill):
    v2 = v.reshape(NW, E_W)
    return jnp.pad(v2, ((0, 0), (0, EP_W - E_W)),
                   constant_values=fill).reshape(NW * EP_W)


def kernel(z, edge_index, W1, b1, prelu_a, W2, att_src, att_dst, b2):
    src = _pad_edges(edge_index[0].astype(jnp.int32), 0)
    dst = _pad_edges(edge_index[1].astype(jnp.int32), TRASH)
    zeros = jnp.zeros((NP, D), jnp.float32)

    sc_deg, sc_pass1, sc_pass2a, sc_pass2b = _sc_kernels()

    degp = sc_deg(dst)
    degp_t = degp.reshape(NW, NP).T[:N]
    xs, dis = _tc1(z, W1, degp_t)
    acc = sc_pass1(xs, src, dst, zeros)
    xg, a_src, a_dst, shift, exself = _tc2(
        acc, xs, dis, b1.reshape(1, D), prelu_a.reshape(1, D), W2,
        att_src.reshape(1, D), att_dst.reshape(1, D))
    asrc_p = jnp.pad(a_src.reshape(N), (0, NP - N))
    adst_p = jnp.pad(a_dst.reshape(N), (0, NP - N))
    ex, denp = sc_pass2a(src, dst, asrc_p, adst_p, shift.reshape(L))
    num = sc_pass2b(xg, src, dst, ex, zeros)
    denp_t = denp.reshape(NW, NP).T[:N]
    out = _tc3(num, xg, denp_t, exself, b2.reshape(1, D))
    return out
